# baseline (speedup 1.0000x reference)
import functools

import jax
import jax.numpy as jnp
from jax import lax
from jax.experimental import pallas as pl
from jax.experimental.pallas import tpu as pltpu
from jax.experimental.pallas import tpu_sc as plsc

VOCAB = 100000
EMBED = 64
BATCH = 1024

_NC, _NS = 2, 16
_NW = _NC * _NS
_BPW = BATCH // _NW

_VB = 4096
_GRID = (VOCAB + _VB - 1) // _VB

_mesh = plsc.VectorSubcoreMesh(
    core_axis_name="c", subcore_axis_name="s", num_cores=_NC, num_subcores=_NS
)


@functools.partial(
    pl.kernel,
    out_type=jax.ShapeDtypeStruct((BATCH, EMBED), jnp.float32),
    mesh=_mesh,
    scratch_types=[
        pltpu.VMEM((_BPW,), jnp.int32),
        pltpu.VMEM((_BPW, EMBED), jnp.float32),
        pltpu.SemaphoreType.DMA,
    ],
    compiler_params=pltpu.CompilerParams(use_tc_tiling_on_sc=False),
)
def _sc_gather(emb_hbm, idx_hbm, out_hbm, idx_v, rows_v, sem):
    wid = lax.axis_index("s") * _NC + lax.axis_index("c")
    base = wid * _BPW
    pltpu.sync_copy(idx_hbm.at[pl.ds(base, _BPW)], idx_v)
    pltpu.async_copy(emb_hbm.at[idx_v], rows_v, sem).wait()
    pltpu.sync_copy(rows_v, out_hbm.at[pl.ds(base, _BPW)])


def _matmul_body(e_ref, w_ref, out_ref):
    e = e_ref[...].astype(jnp.bfloat16)
    w = w_ref[...].astype(jnp.bfloat16)
    out_ref[...] = lax.dot_general(
        e, w, (((1,), (1,)), ((), ())), preferred_element_type=jnp.float32
    )


def kernel(x, emb, W):
    e = _sc_gather(emb, x)
    logits = pl.pallas_call(
        _matmul_body,
        grid=(_GRID,),
        in_specs=[
            pl.BlockSpec((BATCH, EMBED), lambda j: (0, 0)),
            pl.BlockSpec((_VB, EMBED), lambda j: (j, 0)),
        ],
        out_specs=pl.BlockSpec((BATCH, _VB), lambda j: (0, j)),
        out_shape=jax.ShapeDtypeStruct((BATCH, VOCAB), jnp.float32),
    )(e, W)
    return logits, e


# speedup vs baseline: 2.3653x; 2.3653x over previous
import functools

import jax
import jax.numpy as jnp
from jax import lax
from jax.experimental import pallas as pl
from jax.experimental.pallas import tpu as pltpu
from jax.experimental.pallas import tpu_sc as plsc

VOCAB = 100000
EMBED = 64
BATCH = 1024

_NC, _NS = 2, 16
_NW = _NC * _NS
_BPW = BATCH // _NW

_VB = 4096
_GRID = (VOCAB + _VB - 1) // _VB

_mesh = plsc.VectorSubcoreMesh(
    core_axis_name="c", subcore_axis_name="s", num_cores=_NC, num_subcores=_NS
)


@functools.partial(
    pl.kernel,
    out_type=jax.ShapeDtypeStruct((BATCH, EMBED), jnp.float32),
    mesh=_mesh,
    scratch_types=[
        pltpu.VMEM((_BPW,), jnp.int32),
        pltpu.VMEM((_BPW, EMBED), jnp.float32),
        pltpu.SemaphoreType.DMA,
    ],
    compiler_params=pltpu.CompilerParams(use_tc_tiling_on_sc=False),
)
def _sc_gather(emb_hbm, idx_hbm, out_hbm, idx_v, rows_v, sem):
    wid = lax.axis_index("s") * _NC + lax.axis_index("c")
    base = wid * _BPW
    pltpu.sync_copy(idx_hbm.at[pl.ds(base, _BPW)], idx_v)
    pltpu.async_copy(emb_hbm.at[idx_v], rows_v, sem).wait()
    pltpu.sync_copy(rows_v, out_hbm.at[pl.ds(base, _BPW)])


def _matmul_body(w_ref, e_ref, out_ref):
    w = w_ref[...].astype(jnp.bfloat16)
    e = e_ref[...].astype(jnp.bfloat16)
    out_ref[...] = lax.dot_general(
        w, e, (((1,), (1,)), ((), ())), preferred_element_type=jnp.float32
    )


def kernel(x, emb, W):
    e = _sc_gather(emb, x)
    logits_t = pl.pallas_call(
        _matmul_body,
        grid=(_GRID,),
        in_specs=[
            pl.BlockSpec((_VB, EMBED), lambda j: (j, 0)),
            pl.BlockSpec((BATCH, EMBED), lambda j: (0, 0)),
        ],
        out_specs=pl.BlockSpec((_VB, BATCH), lambda j: (j, 0)),
        out_shape=jax.ShapeDtypeStruct((VOCAB, BATCH), jnp.float32),
    )(W, e)
    return logits_t.T, e


# speedup vs baseline: 2.8040x; 1.1855x over previous
import functools

import jax
import jax.numpy as jnp
from jax import lax
from jax.experimental import pallas as pl
from jax.experimental.pallas import tpu as pltpu
from jax.experimental.pallas import tpu_sc as plsc

VOCAB = 100000
EMBED = 64
BATCH = 1024

_NC, _NS = 2, 16
_NW = _NC * _NS
_BPW = BATCH // _NW

_VB = 4096
_GRID = (VOCAB + _VB - 1) // _VB

_mesh = plsc.VectorSubcoreMesh(
    core_axis_name="c", subcore_axis_name="s", num_cores=_NC, num_subcores=_NS
)


@functools.partial(
    pl.kernel,
    out_type=jax.ShapeDtypeStruct((BATCH, EMBED), jnp.float32),
    mesh=_mesh,
    scratch_types=[
        pltpu.VMEM((_BPW,), jnp.int32),
        pltpu.VMEM((_BPW, EMBED), jnp.float32),
        pltpu.SemaphoreType.DMA,
    ],
    compiler_params=pltpu.CompilerParams(use_tc_tiling_on_sc=False),
)
def _sc_gather(emb_hbm, idx_hbm, out_hbm, idx_v, rows_v, sem):
    wid = lax.axis_index("s") * _NC + lax.axis_index("c")
    base = wid * _BPW
    pltpu.sync_copy(idx_hbm.at[pl.ds(base, _BPW)], idx_v)
    pltpu.async_copy(emb_hbm.at[idx_v], rows_v, sem).wait()
    pltpu.sync_copy(rows_v, out_hbm.at[pl.ds(base, _BPW)])


def _matmul_body(wt_ref, e_ref, out_ref):
    wt = wt_ref[...].astype(jnp.bfloat16)
    e = e_ref[...].astype(jnp.bfloat16)
    out_ref[...] = lax.dot_general(
        wt, e, (((0,), (1,)), ((), ())), preferred_element_type=jnp.float32
    )


def kernel(x, emb, W):
    e = _sc_gather(emb, x)
    logits_t = pl.pallas_call(
        _matmul_body,
        grid=(_GRID,),
        in_specs=[
            pl.BlockSpec((EMBED, _VB), lambda j: (0, j)),
            pl.BlockSpec((BATCH, EMBED), lambda j: (0, 0)),
        ],
        out_specs=pl.BlockSpec((_VB, BATCH), lambda j: (j, 0)),
        out_shape=jax.ShapeDtypeStruct((VOCAB, BATCH), jnp.float32),
    )(W.T, e)
    return logits_t.T, e


# speedup vs baseline: 2.9226x; 1.0423x over previous
import functools

import jax
import jax.numpy as jnp
from jax import lax
from jax.experimental import pallas as pl
from jax.experimental.pallas import tpu as pltpu
from jax.experimental.pallas import tpu_sc as plsc

VOCAB = 100000
EMBED = 64
BATCH = 1024

_NC, _NS = 2, 16
_NW = _NC * _NS
_BPW = BATCH // _NW

_VB = 4096
_GRID = (VOCAB + _VB - 1) // _VB

_mesh = plsc.VectorSubcoreMesh(
    core_axis_name="c", subcore_axis_name="s", num_cores=_NC, num_subcores=_NS
)


_DPAD = 128


@functools.partial(
    pl.kernel,
    out_type=jax.ShapeDtypeStruct((BATCH, _DPAD), jnp.float32),
    mesh=_mesh,
    scratch_types=[
        pltpu.VMEM((_BPW,), jnp.int32),
        pltpu.VMEM((_BPW, _DPAD), jnp.float32),
        pltpu.SemaphoreType.DMA,
    ],
    compiler_params=pltpu.CompilerParams(use_tc_tiling_on_sc=False),
)
def _sc_gather(emb_hbm, idx_hbm, out_hbm, idx_v, rows_v, sem):
    wid = lax.axis_index("s") * _NC + lax.axis_index("c")
    base = wid * _BPW
    pltpu.sync_copy(idx_hbm.at[pl.ds(base, _BPW)], idx_v)
    pltpu.async_copy(emb_hbm.at[idx_v], rows_v, sem).wait()
    pltpu.sync_copy(rows_v, out_hbm.at[pl.ds(base, _BPW)])


def _matmul_body(wt_ref, e_ref, out_ref):
    wt = wt_ref[...].astype(jnp.bfloat16)
    e = e_ref[...].astype(jnp.bfloat16)
    out_ref[...] = lax.dot_general(
        wt, e, (((0,), (1,)), ((), ())), preferred_element_type=jnp.float32
    )


def kernel(x, emb, W):
    emb_pad = jnp.pad(emb, ((0, 0), (0, _DPAD - EMBED)))
    e_pad = _sc_gather(emb_pad, x)
    e = e_pad[:, :EMBED]
    logits_t = pl.pallas_call(
        _matmul_body,
        grid=(_GRID,),
        in_specs=[
            pl.BlockSpec((EMBED, _VB), lambda j: (0, j)),
            pl.BlockSpec((BATCH, EMBED), lambda j: (0, 0)),
        ],
        out_specs=pl.BlockSpec((_VB, BATCH), lambda j: (j, 0)),
        out_shape=jax.ShapeDtypeStruct((VOCAB, BATCH), jnp.float32),
    )(W.T, e)
    return logits_t.T, e


# speedup vs baseline: 3.2281x; 1.1045x over previous
import functools

import jax
import jax.numpy as jnp
from jax import lax
from jax.experimental import pallas as pl
from jax.experimental.pallas import tpu as pltpu
from jax.experimental.pallas import tpu_sc as plsc

VOCAB = 100000
EMBED = 64
BATCH = 1024

_NC, _NS = 2, 16
_NW = _NC * _NS
_BPW = BATCH // _NW

_VB = 4096
_GRID = (VOCAB + _VB - 1) // _VB

_mesh = plsc.VectorSubcoreMesh(
    core_axis_name="c", subcore_axis_name="s", num_cores=_NC, num_subcores=_NS
)


@functools.partial(
    pl.kernel,
    out_type=jax.ShapeDtypeStruct((BATCH, EMBED), jnp.float32),
    mesh=_mesh,
    scratch_types=[
        pltpu.VMEM((_BPW,), jnp.int32),
        pltpu.VMEM((_BPW, EMBED), jnp.float32),
        pltpu.SemaphoreType.DMA,
    ],
    compiler_params=pltpu.CompilerParams(use_tc_tiling_on_sc=True),
)
def _sc_gather(emb_hbm, idx_hbm, out_hbm, idx_v, rows_v, sem):
    wid = lax.axis_index("s") * _NC + lax.axis_index("c")
    base = wid * _BPW
    pltpu.sync_copy(idx_hbm.at[pl.ds(base, _BPW)], idx_v)
    copies = []
    for j0 in range(0, _BPW, 16):
        vec = idx_v[pl.ds(j0, 16)]
        for j in range(16):
            copies.append(
                pltpu.make_async_copy(
                    emb_hbm.at[vec[j]], rows_v.at[j0 + j], sem
                )
            )
    for c in copies:
        c.start()
    for c in copies:
        c.wait()
    pltpu.sync_copy(rows_v, out_hbm.at[pl.ds(base, _BPW)])


def _matmul_body(wt_ref, e_ref, out_ref):
    wt = wt_ref[...].astype(jnp.bfloat16)
    e = e_ref[...].astype(jnp.bfloat16)
    out_ref[...] = lax.dot_general(
        wt, e, (((0,), (1,)), ((), ())), preferred_element_type=jnp.float32
    )


def kernel(x, emb, W):
    e = _sc_gather(emb, x)
    logits_t = pl.pallas_call(
        _matmul_body,
        grid=(_GRID,),
        in_specs=[
            pl.BlockSpec((EMBED, _VB), lambda j: (0, j)),
            pl.BlockSpec((BATCH, EMBED), lambda j: (0, 0)),
        ],
        out_specs=pl.BlockSpec((_VB, BATCH), lambda j: (j, 0)),
        out_shape=jax.ShapeDtypeStruct((VOCAB, BATCH), jnp.float32),
    )(W.T, e)
    return logits_t.T, e
